# baseline (speedup 1.0000x reference)
import jax
import jax.numpy as jnp
from jax import lax
from jax.experimental import pallas as pl
from jax.experimental.pallas import tpu as pltpu
from jax.experimental.pallas import tpu_sc as plsc

N_DIMS = 64
SEQ_LEN = 200
BATCH = 4096

_INFO = plsc.get_sparse_core_info()
_NC = _INFO.num_cores
_NS = _INFO.num_subcores
_NW = _NC * _NS

_BB = 128
_ST = SEQ_LEN // 8
_NBUF = 4
_NWB = 2


def _body(ex_hbm, cat_hbm, etab_hbm, ctab_hbm, ptab_hbm, out_hbm,
          p_v, ie0, ie1, ie2, ie3, ic0, ic1, ic2, ic3,
          be0, be1, be2, be3, bc0, bc1, bc2, bc3,
          wt0, wt1,
          si0, si1, si2, si3, sg0, sg1, sg2, sg3, so0, so1):
    ie = (ie0, ie1, ie2, ie3)
    ic = (ic0, ic1, ic2, ic3)
    be = (be0, be1, be2, be3)
    bc = (bc0, bc1, bc2, bc3)
    wt = (wt0, wt1)
    si = (si0, si1, si2, si3)
    sg = (sg0, sg1, sg2, sg3)
    so = (so0, so1)

    wid = lax.axis_index("s") * _NC + lax.axis_index("c")

    pltpu.sync_copy(ptab_hbm, p_v)

    lane = lax.iota(jnp.int32, 16)

    def issue_idx(ci, b):
        st = ci // 8
        sr = ci % 8
        pltpu.async_copy(ex_hbm.at[st, wid, sr], ie[b], si[b])
        pltpu.async_copy(cat_hbm.at[st, wid, sr], ic[b], si[b])

    def wait_idx(ci, b):
        st = ci // 8
        sr = ci % 8
        pltpu.make_async_copy(ex_hbm.at[st, wid, sr], ie[b], si[b]).wait()
        pltpu.make_async_copy(cat_hbm.at[st, wid, sr], ic[b], si[b]).wait()

    def issue_gathers(b):
        pltpu.async_copy(etab_hbm.at[ie[b]], be[b], sg[b])
        pltpu.async_copy(ctab_hbm.at[ic[b]], bc[b], sg[b])

    def wait_gathers(b):
        pltpu.make_async_copy(etab_hbm.at[ie[b]], be[b], sg[b]).wait()
        pltpu.make_async_copy(ctab_hbm.at[ic[b]], bc[b], sg[b]).wait()

    def issue_writeback(ci, w):
        for dt in range(8):
            pltpu.async_copy(wt[w].at[pl.ds(dt * 8, 8), pl.ds(0, _BB)],
                             out_hbm.at[ci, dt, wid], so[w])

    def wait_writeback(ci, w):
        for dt in range(8):
            pltpu.make_async_copy(wt[w].at[pl.ds(dt * 8, 8), pl.ds(0, _BB)],
                                  out_hbm.at[ci, dt, wid], so[w]).wait()

    for b in range(_NBUF):
        issue_idx(b, b)
    for b in range(_NBUF - 1):
        wait_idx(b, b)
        issue_gathers(b)

    def chunk(ci, b, w):
        wait_gathers(b)

        @pl.when(ci + _NBUF < SEQ_LEN)
        def _():
            issue_idx(ci + _NBUF, b)

        fi = ci + _NBUF - 1
        fb = (b + _NBUF - 1) % _NBUF

        @pl.when(fi < SEQ_LEN)
        def _():
            wait_idx(fi, fb)
            issue_gathers(fb)

        @pl.when(ci >= _NWB)
        def _():
            wait_writeback(ci - _NWB, w)

        pvec = [p_v[ci, pl.ds(k * 16, 16)] for k in range(N_DIMS // 16)]
        dvec = [lane + k * 16 for k in range(N_DIMS // 16)]

        @plsc.parallel_loop(0, _BB, unroll=8)
        def row_body(j):
            jv = jnp.full((16,), j, dtype=jnp.int32)
            for k in range(N_DIMS // 16):
                sl = pl.ds(k * 16, 16)
                v = be[b][j, sl] + bc[b][j, sl] + pvec[k]
                plsc.store_scatter(wt[w], [dvec[k], jv], v)
        issue_writeback(ci, w)

    def outer(g, carry):
        for b in range(_NBUF):
            ci = g * _NBUF + b
            chunk(ci, b, b % _NWB)
        return carry

    lax.fori_loop(0, SEQ_LEN // _NBUF, outer, 0)

    wait_writeback(SEQ_LEN - 2, 0)
    wait_writeback(SEQ_LEN - 1, 1)


@jax.jit
def _run(ex4, cat4, etab, ctab, ptab):
    mesh = plsc.VectorSubcoreMesh(core_axis_name="c", subcore_axis_name="s")
    f = pl.kernel(
        _body,
        out_type=jax.ShapeDtypeStruct((SEQ_LEN, 8, _NW, 8, _BB), jnp.float32),
        mesh=mesh,
        scratch_types=(
            [pltpu.VMEM((SEQ_LEN, N_DIMS), jnp.float32)]
            + [pltpu.VMEM((_BB,), jnp.int32)] * (2 * _NBUF)
            + [pltpu.VMEM((_BB, N_DIMS), jnp.float32)] * (2 * _NBUF)
            + [pltpu.VMEM((N_DIMS, _BB + 1), jnp.float32)] * _NWB
            + [pltpu.SemaphoreType.DMA] * (2 * _NBUF + _NWB)
        ),
        compiler_params=pltpu.CompilerParams(use_tc_tiling_on_sc=False,
                                             needs_layout_passes=False),
    )
    return f(ex4, cat4, etab, ctab, ptab)


def _to_native4d(idx2d):
    return (idx2d.astype(jnp.int32).T
            .reshape(_ST, 8, _NW, _BB)
            .transpose(0, 2, 1, 3))


def kernel(exercises, categories, exercise_table, category_table, position_table):
    out5 = _run(_to_native4d(exercises), _to_native4d(categories),
                exercise_table, category_table, position_table)
    return (out5.transpose(2, 4, 0, 1, 3)
            .reshape(BATCH, SEQ_LEN, N_DIMS))


# speedup vs baseline: 1.0228x; 1.0228x over previous
import jax
import jax.numpy as jnp
from jax import lax
from jax.experimental import pallas as pl
from jax.experimental.pallas import tpu as pltpu
from jax.experimental.pallas import tpu_sc as plsc

N_DIMS = 64
SEQ_LEN = 200
BATCH = 4096

_INFO = plsc.get_sparse_core_info()
_NC = _INFO.num_cores
_NS = _INFO.num_subcores
_NW = _NC * _NS

_BB = 128
_ST = SEQ_LEN // 8
_NBUF = 4
_NWB = 2


def _body(ex_hbm, cat_hbm, etab_hbm, ctab_hbm, ptab_hbm, out_hbm,
          p_v, ie0, ie1, ie2, ie3, ic0, ic1, ic2, ic3,
          be0, be1, be2, be3, bc0, bc1, bc2, bc3,
          wt0, wt1,
          si0, si1, si2, si3, sg0, sg1, sg2, sg3, so0, so1):
    ie = (ie0, ie1, ie2, ie3)
    ic = (ic0, ic1, ic2, ic3)
    be = (be0, be1, be2, be3)
    bc = (bc0, bc1, bc2, bc3)
    wt = (wt0, wt1)
    si = (si0, si1, si2, si3)
    sg = (sg0, sg1, sg2, sg3)
    so = (so0, so1)

    wid = lax.axis_index("s") * _NC + lax.axis_index("c")

    pltpu.sync_copy(ptab_hbm, p_v)

    lane = lax.iota(jnp.int32, 16)

    def issue_idx(ci, b):
        st = ci // 8
        sr = ci % 8
        pltpu.async_copy(ex_hbm.at[st, wid, sr], ie[b], si[b])
        pltpu.async_copy(cat_hbm.at[st, wid, sr], ic[b], si[b])

    def wait_idx(ci, b):
        st = ci // 8
        sr = ci % 8
        pltpu.make_async_copy(ex_hbm.at[st, wid, sr], ie[b], si[b]).wait()
        pltpu.make_async_copy(cat_hbm.at[st, wid, sr], ic[b], si[b]).wait()

    def issue_gathers(b):
        pltpu.async_copy(etab_hbm.at[ie[b]], be[b], sg[b])
        pltpu.async_copy(ctab_hbm.at[ic[b]], bc[b], sg[b])

    def wait_gathers(b):
        pltpu.make_async_copy(etab_hbm.at[ie[b]], be[b], sg[b]).wait()
        pltpu.make_async_copy(ctab_hbm.at[ic[b]], bc[b], sg[b]).wait()

    def issue_writeback(ci, w):
        for dt in range(8):
            pltpu.async_copy(wt[w].at[pl.ds(dt * 8, 8), pl.ds(0, _BB)],
                             out_hbm.at[ci, dt, wid], so[w])

    def wait_writeback(ci, w):
        for dt in range(8):
            pltpu.make_async_copy(wt[w].at[pl.ds(dt * 8, 8), pl.ds(0, _BB)],
                                  out_hbm.at[ci, dt, wid], so[w]).wait()

    for b in range(_NBUF):
        issue_idx(b, b)
    for b in range(_NBUF - 1):
        wait_idx(b, b)
        issue_gathers(b)

    def chunk(ci, b, w):
        wait_gathers(b)

        @pl.when(ci + _NBUF < SEQ_LEN)
        def _():
            issue_idx(ci + _NBUF, b)

        fi = ci + _NBUF - 1
        fb = (b + _NBUF - 1) % _NBUF

        @pl.when(fi < SEQ_LEN)
        def _():
            wait_idx(fi, fb)
            issue_gathers(fb)

        @pl.when(ci >= _NWB)
        def _():
            wait_writeback(ci - _NWB, w)

        pvec = [p_v[ci, pl.ds(k * 16, 16)] for k in range(N_DIMS // 16)]
        dvec = [lane + k * 16 for k in range(N_DIMS // 16)]

        @plsc.parallel_loop(0, _BB, unroll=4)
        def row_body(j):
            jv = jnp.full((16,), j, dtype=jnp.int32)
            for k in range(N_DIMS // 16):
                sl = pl.ds(k * 16, 16)
                v = be[b][j, sl] + bc[b][j, sl] + pvec[k]
                plsc.store_scatter(wt[w], [dvec[k], jv], v)
        issue_writeback(ci, w)

    def outer(g, carry):
        for b in range(_NBUF):
            ci = g * _NBUF + b
            chunk(ci, b, b % _NWB)
        return carry

    lax.fori_loop(0, SEQ_LEN // _NBUF, outer, 0)

    wait_writeback(SEQ_LEN - 2, 0)
    wait_writeback(SEQ_LEN - 1, 1)


@jax.jit
def _run(ex4, cat4, etab, ctab, ptab):
    mesh = plsc.VectorSubcoreMesh(core_axis_name="c", subcore_axis_name="s")
    f = pl.kernel(
        _body,
        out_type=jax.ShapeDtypeStruct((SEQ_LEN, 8, _NW, 8, _BB), jnp.float32),
        mesh=mesh,
        scratch_types=(
            [pltpu.VMEM((SEQ_LEN, N_DIMS), jnp.float32)]
            + [pltpu.VMEM((_BB,), jnp.int32)] * (2 * _NBUF)
            + [pltpu.VMEM((_BB, N_DIMS), jnp.float32)] * (2 * _NBUF)
            + [pltpu.VMEM((N_DIMS, _BB + 1), jnp.float32)] * _NWB
            + [pltpu.SemaphoreType.DMA] * (2 * _NBUF + _NWB)
        ),
        compiler_params=pltpu.CompilerParams(use_tc_tiling_on_sc=False,
                                             needs_layout_passes=False),
    )
    return f(ex4, cat4, etab, ctab, ptab)


def _to_native4d(idx2d):
    return (idx2d.astype(jnp.int32).T
            .reshape(_ST, 8, _NW, _BB)
            .transpose(0, 2, 1, 3))


def kernel(exercises, categories, exercise_table, category_table, position_table):
    out5 = _run(_to_native4d(exercises), _to_native4d(categories),
                exercise_table, category_table, position_table)
    return (out5.transpose(2, 4, 0, 1, 3)
            .reshape(BATCH, SEQ_LEN, N_DIMS))
